# Initial kernel scaffold; baseline (speedup 1.0000x reference)
import jax
import jax.numpy as jnp
from jax.experimental import pallas as pl


def kernel(inputs, state, gate_weights, gate_biases, candidate_weights, candidate_biases, sup_rows, sup_cols, sup_vals):
    raise NotImplementedError("write your pallas kernel here")


# speedup vs baseline: 372.7090x; 372.7090x over previous
import jax
import jax.numpy as jnp
from jax.experimental import pallas as pl

N = 512
U = 128
D_IN = 128
IN_SZ = D_IN + U
NM = 3


def _cell_kernel(s_ref, inp_ref, st_ref, wg_ref, bg_ref, wc_ref, bc_ref, out_ref):
    s = s_ref[...]
    inp = inp_ref[0]
    st = st_ref[0]
    ias = jnp.concatenate([inp, st], axis=1)
    x1 = jnp.dot(s, ias, preferred_element_type=jnp.float32)
    x2 = 2.0 * jnp.dot(s, x1, preferred_element_type=jnp.float32) - ias
    g = (
        jnp.dot(ias, wg_ref[0], preferred_element_type=jnp.float32)
        + jnp.dot(x1, wg_ref[1], preferred_element_type=jnp.float32)
        + jnp.dot(x2, wg_ref[2], preferred_element_type=jnp.float32)
        + bg_ref[0]
    )
    g = jax.nn.sigmoid(g)
    r = g[:, :U]
    u = g[:, U:]
    ias2 = jnp.concatenate([inp, r * st], axis=1)
    y1 = jnp.dot(s, ias2, preferred_element_type=jnp.float32)
    y2 = 2.0 * jnp.dot(s, y1, preferred_element_type=jnp.float32) - ias2
    c = (
        jnp.dot(ias2, wc_ref[0], preferred_element_type=jnp.float32)
        + jnp.dot(y1, wc_ref[1], preferred_element_type=jnp.float32)
        + jnp.dot(y2, wc_ref[2], preferred_element_type=jnp.float32)
        + bc_ref[0]
    )
    c = jnp.tanh(c)
    out_ref[0] = u * st + (1.0 - u) * c


def kernel(inputs, state, gate_weights, gate_biases, candidate_weights, candidate_biases, sup_rows, sup_cols, sup_vals):
    B = inputs.shape[0]
    inp = inputs.reshape(B, N, D_IN)
    st = state.reshape(B, N, U)
    wg = gate_weights.reshape(IN_SZ, NM, 2 * U).transpose(1, 0, 2)
    wc = candidate_weights.reshape(IN_SZ, NM, U).transpose(1, 0, 2)
    bg = gate_biases.reshape(1, 2 * U)
    bc = candidate_biases.reshape(1, U)

    s_dense = jnp.zeros((N, N), jnp.float32).at[sup_rows, sup_cols].set(sup_vals)

    out = pl.pallas_call(
        _cell_kernel,
        grid=(B,),
        in_specs=[
            pl.BlockSpec((N, N), lambda b: (0, 0)),
            pl.BlockSpec((1, N, D_IN), lambda b: (b, 0, 0)),
            pl.BlockSpec((1, N, U), lambda b: (b, 0, 0)),
            pl.BlockSpec((NM, IN_SZ, 2 * U), lambda b: (0, 0, 0)),
            pl.BlockSpec((1, 2 * U), lambda b: (0, 0)),
            pl.BlockSpec((NM, IN_SZ, U), lambda b: (0, 0, 0)),
            pl.BlockSpec((1, U), lambda b: (0, 0)),
        ],
        out_specs=pl.BlockSpec((1, N, U), lambda b: (b, 0, 0)),
        out_shape=jax.ShapeDtypeStruct((B, N, U), jnp.float32),
    )(s_dense, inp, st, wg, bg, wc, bc)

    y = out.reshape(B, N * U)
    return y, y


# speedup vs baseline: 498.1679x; 1.3366x over previous
import functools

import jax
import jax.numpy as jnp
from jax import lax
from jax.experimental import pallas as pl
from jax.experimental.pallas import tpu as pltpu
from jax.experimental.pallas import tpu_sc as plsc

N = 512
U = 128
D_IN = 128
IN_SZ = D_IN + U
NM = 3

LANES = 16
NW = 32
ROWS_PER_W = N // NW
TILE_WORDS = ROWS_PER_W * N


def _densify_body(nnzp, sr_hbm, sc_hbm, sv_hbm, out_hbm, sr_v, sc_v, sv_v, tile_v):
    wid = lax.axis_index("s") * 2 + lax.axis_index("c")
    lo = wid * ROWS_PER_W
    pltpu.sync_copy(sr_hbm, sr_v)
    pltpu.sync_copy(sc_hbm, sc_v)
    pltpu.sync_copy(sv_hbm, sv_v)

    zeros16 = jnp.zeros((LANES,), jnp.float32)

    def zero_body(i, _):
        tile_v[pl.ds(i * LANES, LANES)] = zeros16
        return 0

    lax.fori_loop(0, TILE_WORDS // LANES, zero_body, 0)

    def edge_body(i, _):
        r = sr_v[pl.ds(i * LANES, LANES)]
        c = sc_v[pl.ds(i * LANES, LANES)]
        v = sv_v[pl.ds(i * LANES, LANES)]
        m = (r >= lo) & (r < lo + ROWS_PER_W)
        lin = (r - lo) * N + c
        plsc.store_scatter(tile_v, [lin], v, mask=m)
        return 0

    lax.fori_loop(0, nnzp // LANES, edge_body, 0)
    pltpu.sync_copy(tile_v, out_hbm.at[pl.ds(wid * TILE_WORDS, TILE_WORDS)])


def _densify(sup_rows, sup_cols, sup_vals):
    nnz = sup_rows.shape[0]
    nnzp = -(-nnz // LANES) * LANES
    pad = nnzp - nnz
    sr = jnp.concatenate([sup_rows.astype(jnp.int32), jnp.full((pad,), 2 * N, jnp.int32)])
    sc = jnp.concatenate([sup_cols.astype(jnp.int32), jnp.zeros((pad,), jnp.int32)])
    sv = jnp.concatenate([sup_vals, jnp.zeros((pad,), jnp.float32)])

    mesh = plsc.VectorSubcoreMesh(core_axis_name="c", subcore_axis_name="s")
    fn = functools.partial(
        pl.kernel,
        mesh=mesh,
        out_type=jax.ShapeDtypeStruct((N * N,), jnp.float32),
        scratch_types=[
            pltpu.VMEM((nnzp,), jnp.int32),
            pltpu.VMEM((nnzp,), jnp.int32),
            pltpu.VMEM((nnzp,), jnp.float32),
            pltpu.VMEM((TILE_WORDS,), jnp.float32),
        ],
        compiler_params=pltpu.CompilerParams(needs_layout_passes=False),
    )(functools.partial(_densify_body, nnzp))
    return fn(sr, sc, sv).reshape(N, N)


def _cell_kernel(s_ref, inp_ref, st_ref, wg_ref, bg_ref, wc_ref, bc_ref, out_ref):
    s = s_ref[...]
    inp = inp_ref[0]
    st = st_ref[0]
    ias = jnp.concatenate([inp, st], axis=1)
    x1 = jnp.dot(s, ias, preferred_element_type=jnp.float32)
    x2 = 2.0 * jnp.dot(s, x1, preferred_element_type=jnp.float32) - ias
    g = (
        jnp.dot(ias, wg_ref[0], preferred_element_type=jnp.float32)
        + jnp.dot(x1, wg_ref[1], preferred_element_type=jnp.float32)
        + jnp.dot(x2, wg_ref[2], preferred_element_type=jnp.float32)
        + bg_ref[0]
    )
    g = jax.nn.sigmoid(g)
    r = g[:, :U]
    u = g[:, U:]
    ias2 = jnp.concatenate([inp, r * st], axis=1)
    y1 = jnp.dot(s, ias2, preferred_element_type=jnp.float32)
    y2 = 2.0 * jnp.dot(s, y1, preferred_element_type=jnp.float32) - ias2
    c = (
        jnp.dot(ias2, wc_ref[0], preferred_element_type=jnp.float32)
        + jnp.dot(y1, wc_ref[1], preferred_element_type=jnp.float32)
        + jnp.dot(y2, wc_ref[2], preferred_element_type=jnp.float32)
        + bc_ref[0]
    )
    c = jnp.tanh(c)
    out_ref[0] = u * st + (1.0 - u) * c


def kernel(inputs, state, gate_weights, gate_biases, candidate_weights, candidate_biases, sup_rows, sup_cols, sup_vals):
    B = inputs.shape[0]
    inp = inputs.reshape(B, N, D_IN)
    st = state.reshape(B, N, U)
    wg = gate_weights.reshape(IN_SZ, NM, 2 * U).transpose(1, 0, 2)
    wc = candidate_weights.reshape(IN_SZ, NM, U).transpose(1, 0, 2)
    bg = gate_biases.reshape(1, 2 * U)
    bc = candidate_biases.reshape(1, U)

    s_dense = _densify(sup_rows, sup_cols, sup_vals)

    out = pl.pallas_call(
        _cell_kernel,
        grid=(B,),
        in_specs=[
            pl.BlockSpec((N, N), lambda b: (0, 0)),
            pl.BlockSpec((1, N, D_IN), lambda b: (b, 0, 0)),
            pl.BlockSpec((1, N, U), lambda b: (b, 0, 0)),
            pl.BlockSpec((NM, IN_SZ, 2 * U), lambda b: (0, 0, 0)),
            pl.BlockSpec((1, 2 * U), lambda b: (0, 0)),
            pl.BlockSpec((NM, IN_SZ, U), lambda b: (0, 0, 0)),
            pl.BlockSpec((1, U), lambda b: (0, 0)),
        ],
        out_specs=pl.BlockSpec((1, N, U), lambda b: (b, 0, 0)),
        out_shape=jax.ShapeDtypeStruct((B, N, U), jnp.float32),
    )(s_dense, inp, st, wg, bg, wc, bc)

    y = out.reshape(B, N * U)
    return y, y


# speedup vs baseline: 540.0899x; 1.0842x over previous
import functools

import jax
import jax.numpy as jnp
from jax import lax
from jax.experimental import pallas as pl
from jax.experimental.pallas import tpu as pltpu
from jax.experimental.pallas import tpu_sc as plsc

N = 512
U = 128
D_IN = 128
IN_SZ = D_IN + U
NM = 3

LANES = 16
NW = 32
ROWS_PER_W = N // NW
TILE_WORDS = ROWS_PER_W * N


def _densify_body(nnz, sr_hbm, sc_hbm, sv_hbm, out_hbm, sr_v, sc_v, sv_v, tile_v):
    wid = lax.axis_index("s") * 2 + lax.axis_index("c")
    lo = wid * ROWS_PER_W
    pltpu.sync_copy(sr_hbm, sr_v.at[pl.ds(0, nnz)])
    pltpu.sync_copy(sc_hbm, sc_v.at[pl.ds(0, nnz)])
    pltpu.sync_copy(sv_hbm, sv_v.at[pl.ds(0, nnz)])

    zeros16 = jnp.zeros((LANES,), jnp.float32)

    def zero_body(i, _):
        tile_v[pl.ds(i * LANES, LANES)] = zeros16
        return 0

    lax.fori_loop(0, TILE_WORDS // LANES, zero_body, 0)

    def lower_bound(target):
        def cond(c):
            return c[0] < c[1]

        def body(c):
            lb, ub = c
            mid = (lb + ub) // 2
            v = sr_v[pl.ds(mid, LANES)][0]
            lt = v < target
            return jnp.where(lt, mid + 1, lb), jnp.where(lt, ub, mid)

        return lax.while_loop(cond, body, (jnp.int32(0), jnp.int32(nnz)))[0]

    e0 = lower_bound(lo)
    e1 = lower_bound(lo + ROWS_PER_W)
    start16 = (e0 // LANES) * LANES
    nvec = (e1 - start16 + LANES - 1) // LANES
    lane = lax.iota(jnp.int32, LANES)

    def edge_body(k, _):
        base = start16 + k * LANES
        r = sr_v[pl.ds(base, LANES)]
        c = sc_v[pl.ds(base, LANES)]
        v = sv_v[pl.ds(base, LANES)]
        m = (r >= lo) & (r < lo + ROWS_PER_W) & (base + lane < nnz)
        lin = (r - lo) * N + c
        plsc.store_scatter(tile_v, [lin], v, mask=m)
        return 0

    lax.fori_loop(0, nvec, edge_body, 0)
    pltpu.sync_copy(tile_v, out_hbm.at[pl.ds(wid * TILE_WORDS, TILE_WORDS)])


def _densify(sup_rows, sup_cols, sup_vals):
    nnz = sup_rows.shape[0]
    nbuf = (nnz // LANES + 2) * LANES

    mesh = plsc.VectorSubcoreMesh(core_axis_name="c", subcore_axis_name="s")
    fn = functools.partial(
        pl.kernel,
        mesh=mesh,
        out_type=jax.ShapeDtypeStruct((N * N,), jnp.float32),
        scratch_types=[
            pltpu.VMEM((nbuf,), jnp.int32),
            pltpu.VMEM((nbuf,), jnp.int32),
            pltpu.VMEM((nbuf,), jnp.float32),
            pltpu.VMEM((TILE_WORDS,), jnp.float32),
        ],
        compiler_params=pltpu.CompilerParams(needs_layout_passes=False),
    )(functools.partial(_densify_body, nnz))
    return fn(sup_rows.astype(jnp.int32), sup_cols.astype(jnp.int32), sup_vals).reshape(N, N)


def _cell_kernel(s_ref, inp_ref, st_ref, wg_ref, bg_ref, wc_ref, bc_ref, out_ref):
    s = s_ref[...]
    inp = inp_ref[0]
    st = st_ref[0]
    ias = jnp.concatenate([inp, st], axis=1)
    x1 = jnp.dot(s, ias, preferred_element_type=jnp.float32)
    x2 = 2.0 * jnp.dot(s, x1, preferred_element_type=jnp.float32) - ias
    g = (
        jnp.dot(ias, wg_ref[0], preferred_element_type=jnp.float32)
        + jnp.dot(x1, wg_ref[1], preferred_element_type=jnp.float32)
        + jnp.dot(x2, wg_ref[2], preferred_element_type=jnp.float32)
        + bg_ref[0]
    )
    g = jax.nn.sigmoid(g)
    r = g[:, :U]
    u = g[:, U:]
    ias2 = jnp.concatenate([inp, r * st], axis=1)
    y1 = jnp.dot(s, ias2, preferred_element_type=jnp.float32)
    y2 = 2.0 * jnp.dot(s, y1, preferred_element_type=jnp.float32) - ias2
    c = (
        jnp.dot(ias2, wc_ref[0], preferred_element_type=jnp.float32)
        + jnp.dot(y1, wc_ref[1], preferred_element_type=jnp.float32)
        + jnp.dot(y2, wc_ref[2], preferred_element_type=jnp.float32)
        + bc_ref[0]
    )
    c = jnp.tanh(c)
    out_ref[0] = u * st + (1.0 - u) * c


def kernel(inputs, state, gate_weights, gate_biases, candidate_weights, candidate_biases, sup_rows, sup_cols, sup_vals):
    B = inputs.shape[0]
    inp = inputs.reshape(B, N, D_IN)
    st = state.reshape(B, N, U)
    wg = gate_weights.reshape(IN_SZ, NM, 2 * U).transpose(1, 0, 2)
    wc = candidate_weights.reshape(IN_SZ, NM, U).transpose(1, 0, 2)
    bg = gate_biases.reshape(1, 2 * U)
    bc = candidate_biases.reshape(1, U)

    s_dense = _densify(sup_rows, sup_cols, sup_vals)

    out = pl.pallas_call(
        _cell_kernel,
        grid=(B,),
        in_specs=[
            pl.BlockSpec((N, N), lambda b: (0, 0)),
            pl.BlockSpec((1, N, D_IN), lambda b: (b, 0, 0)),
            pl.BlockSpec((1, N, U), lambda b: (b, 0, 0)),
            pl.BlockSpec((NM, IN_SZ, 2 * U), lambda b: (0, 0, 0)),
            pl.BlockSpec((1, 2 * U), lambda b: (0, 0)),
            pl.BlockSpec((NM, IN_SZ, U), lambda b: (0, 0, 0)),
            pl.BlockSpec((1, U), lambda b: (0, 0)),
        ],
        out_specs=pl.BlockSpec((1, N, U), lambda b: (b, 0, 0)),
        out_shape=jax.ShapeDtypeStruct((B, N, U), jnp.float32),
    )(s_dense, inp, st, wg, bg, wc, bc)

    y = out.reshape(B, N * U)
    return y, y


# speedup vs baseline: 568.1673x; 1.0520x over previous
import functools

import jax
import jax.numpy as jnp
from jax import lax
from jax.experimental import pallas as pl
from jax.experimental.pallas import tpu as pltpu
from jax.experimental.pallas import tpu_sc as plsc

N = 512
U = 128
D_IN = 128
IN_SZ = D_IN + U
NM = 3

LANES = 16
NW = 32
ROWS_PER_W = N // NW
TILE_WORDS = ROWS_PER_W * N


def _densify_body(nnz, sr_hbm, sc_hbm, sv_hbm, out_hbm, sr_v, sc_v, sv_v, tile_v):
    wid = lax.axis_index("s") * 2 + lax.axis_index("c")
    lo = wid * ROWS_PER_W
    pltpu.sync_copy(sr_hbm, sr_v.at[pl.ds(0, nnz)])
    pltpu.sync_copy(sc_hbm, sc_v.at[pl.ds(0, nnz)])
    pltpu.sync_copy(sv_hbm, sv_v.at[pl.ds(0, nnz)])

    zeros16 = jnp.zeros((LANES,), jnp.float32)

    def zero_body(i, _):
        tile_v[pl.ds(i * LANES, LANES)] = zeros16
        return 0

    lax.fori_loop(0, TILE_WORDS // LANES, zero_body, 0)

    def lower_bound(target):
        def cond(c):
            return c[0] < c[1]

        def body(c):
            lb, ub = c
            mid = (lb + ub) // 2
            v = sr_v[pl.ds(mid, LANES)][0]
            lt = v < target
            return jnp.where(lt, mid + 1, lb), jnp.where(lt, ub, mid)

        return lax.while_loop(cond, body, (jnp.int32(0), jnp.int32(nnz)))[0]

    e0 = lower_bound(lo)
    e1 = lower_bound(lo + ROWS_PER_W)
    start16 = (e0 // LANES) * LANES
    nvec = (e1 - start16 + LANES - 1) // LANES
    lane = lax.iota(jnp.int32, LANES)

    def edge_body(k, _):
        base = start16 + k * LANES
        r = sr_v[pl.ds(base, LANES)]
        c = sc_v[pl.ds(base, LANES)]
        v = sv_v[pl.ds(base, LANES)]
        m = (r >= lo) & (r < lo + ROWS_PER_W) & (base + lane < nnz)
        lin = (r - lo) * N + c
        plsc.store_scatter(tile_v, [lin], v, mask=m)
        return 0

    lax.fori_loop(0, nvec, edge_body, 0)
    pltpu.sync_copy(tile_v, out_hbm.at[pl.ds(wid * TILE_WORDS, TILE_WORDS)])


def _densify(sup_rows, sup_cols, sup_vals):
    nnz = sup_rows.shape[0]
    nbuf = (nnz // LANES + 2) * LANES

    mesh = plsc.VectorSubcoreMesh(core_axis_name="c", subcore_axis_name="s")
    fn = functools.partial(
        pl.kernel,
        mesh=mesh,
        out_type=jax.ShapeDtypeStruct((N * N,), jnp.float32),
        scratch_types=[
            pltpu.VMEM((nbuf,), jnp.int32),
            pltpu.VMEM((nbuf,), jnp.int32),
            pltpu.VMEM((nbuf,), jnp.float32),
            pltpu.VMEM((TILE_WORDS,), jnp.float32),
        ],
        compiler_params=pltpu.CompilerParams(needs_layout_passes=False),
    )(functools.partial(_densify_body, nnz))
    return fn(sup_rows.astype(jnp.int32), sup_cols.astype(jnp.int32), sup_vals).reshape(N, N)


def _dot(a, b):
    return jnp.dot(a, b, preferred_element_type=jnp.float32)


def _cell_kernel(s_ref, inp_ref, st_ref, wg_ref, bg_ref, wc_ref, bc_ref, out_ref, out2_ref):
    s = s_ref[...]
    i0 = inp_ref[0]
    i1 = inp_ref[1]
    s0 = st_ref[0]
    s1 = st_ref[1]
    ist = jnp.concatenate([i0, i1, s0, s1], axis=1)
    d1 = _dot(s, ist)
    d2 = _dot(s, d1)
    g_in = jnp.concatenate(
        [
            jnp.concatenate([i0, s0, d1[:, 0:128], d1[:, 256:384], d2[:, 0:128], d2[:, 256:384]], axis=1),
            jnp.concatenate([i1, s1, d1[:, 128:256], d1[:, 384:512], d2[:, 128:256], d2[:, 384:512]], axis=1),
        ],
        axis=0,
    )
    g = jax.nn.sigmoid(_dot(g_in, wg_ref[...]) + bg_ref[0])
    r0 = g[:N, :U]
    u0 = g[:N, U:]
    r1 = g[N:, :U]
    u1 = g[N:, U:]
    rs0 = r0 * s0
    rs1 = r1 * s1
    e1 = _dot(s, jnp.concatenate([rs0, rs1], axis=1))
    e2 = _dot(s, e1)
    c_in = jnp.concatenate(
        [
            jnp.concatenate([i0, rs0, d1[:, 0:128], e1[:, 0:128], d2[:, 0:128], e2[:, 0:128]], axis=1),
            jnp.concatenate([i1, rs1, d1[:, 128:256], e1[:, 128:256], d2[:, 128:256], e2[:, 128:256]], axis=1),
        ],
        axis=0,
    )
    c = jnp.tanh(_dot(c_in, wc_ref[...]) + bc_ref[0])
    o0 = u0 * s0 + (1.0 - u0) * c[:N]
    o1 = u1 * s1 + (1.0 - u1) * c[N:]
    out_ref[0] = o0
    out_ref[1] = o1
    out2_ref[0] = o0
    out2_ref[1] = o1


def _prep_weights(w, out_sz):
    w3 = w.reshape(IN_SZ, NM, out_sz)
    return jnp.concatenate(
        [
            w3[:D_IN, 0] - w3[:D_IN, 2],
            w3[D_IN:, 0] - w3[D_IN:, 2],
            w3[:D_IN, 1],
            w3[D_IN:, 1],
            2.0 * w3[:D_IN, 2],
            2.0 * w3[D_IN:, 2],
        ],
        axis=0,
    )


def kernel(inputs, state, gate_weights, gate_biases, candidate_weights, candidate_biases, sup_rows, sup_cols, sup_vals):
    B = inputs.shape[0]
    BB = 2
    inp = inputs.reshape(B, N, D_IN)
    st = state.reshape(B, N, U)
    wg = _prep_weights(gate_weights, 2 * U)
    wc = _prep_weights(candidate_weights, U)
    bg = gate_biases.reshape(1, 2 * U)
    bc = candidate_biases.reshape(1, U)

    s_dense = _densify(sup_rows, sup_cols, sup_vals)

    out, out2 = pl.pallas_call(
        _cell_kernel,
        grid=(B // BB,),
        in_specs=[
            pl.BlockSpec((N, N), lambda b: (0, 0)),
            pl.BlockSpec((BB, N, D_IN), lambda b: (b, 0, 0)),
            pl.BlockSpec((BB, N, U), lambda b: (b, 0, 0)),
            pl.BlockSpec((NM * IN_SZ, 2 * U), lambda b: (0, 0)),
            pl.BlockSpec((1, 2 * U), lambda b: (0, 0)),
            pl.BlockSpec((NM * IN_SZ, U), lambda b: (0, 0)),
            pl.BlockSpec((1, U), lambda b: (0, 0)),
        ],
        out_specs=[
            pl.BlockSpec((BB, N, U), lambda b: (b, 0, 0)),
            pl.BlockSpec((BB, N, U), lambda b: (b, 0, 0)),
        ],
        out_shape=[
            jax.ShapeDtypeStruct((B, N, U), jnp.float32),
            jax.ShapeDtypeStruct((B, N, U), jnp.float32),
        ],
    )(s_dense, inp, st, wg, bg, wc, bc)

    return out.reshape(B, N * U), out2.reshape(B, N * U)


# speedup vs baseline: 568.4335x; 1.0005x over previous
import functools

import jax
import jax.numpy as jnp
from jax import lax
from jax.experimental import pallas as pl
from jax.experimental.pallas import tpu as pltpu
from jax.experimental.pallas import tpu_sc as plsc

N = 512
U = 128
D_IN = 128
IN_SZ = D_IN + U
NM = 3

LANES = 16
NW = 32
ROWS_PER_W = N // NW
TILE_WORDS = ROWS_PER_W * N


def _densify_body(nnz, sr_hbm, sc_hbm, sv_hbm, out_hbm, sr_v, sc_v, sv_v, tile_v):
    wid = lax.axis_index("s") * 2 + lax.axis_index("c")
    lo = wid * ROWS_PER_W
    pltpu.sync_copy(sr_hbm, sr_v.at[pl.ds(0, nnz)])
    pltpu.sync_copy(sc_hbm, sc_v.at[pl.ds(0, nnz)])
    pltpu.sync_copy(sv_hbm, sv_v.at[pl.ds(0, nnz)])

    zeros16 = jnp.zeros((LANES,), jnp.float32)

    def zero_body(i, _):
        tile_v[pl.ds(i * LANES, LANES)] = zeros16
        return 0

    lax.fori_loop(0, TILE_WORDS // LANES, zero_body, 0)

    def lower_bound(target):
        def cond(c):
            return c[0] < c[1]

        def body(c):
            lb, ub = c
            mid = (lb + ub) // 2
            v = sr_v[pl.ds(mid, LANES)][0]
            lt = v < target
            return jnp.where(lt, mid + 1, lb), jnp.where(lt, ub, mid)

        return lax.while_loop(cond, body, (jnp.int32(0), jnp.int32(nnz)))[0]

    e0 = lower_bound(lo)
    e1 = lower_bound(lo + ROWS_PER_W)
    start16 = (e0 // LANES) * LANES
    nvec = (e1 - start16 + LANES - 1) // LANES
    lane = lax.iota(jnp.int32, LANES)

    def edge_body(k, _):
        base = start16 + k * LANES
        r = sr_v[pl.ds(base, LANES)]
        c = sc_v[pl.ds(base, LANES)]
        v = sv_v[pl.ds(base, LANES)]
        m = (r >= lo) & (r < lo + ROWS_PER_W) & (base + lane < nnz)
        lin = (r - lo) * N + c
        plsc.store_scatter(tile_v, [lin], v, mask=m)
        return 0

    lax.fori_loop(0, nvec, edge_body, 0)
    pltpu.sync_copy(tile_v, out_hbm.at[pl.ds(wid * TILE_WORDS, TILE_WORDS)])


def _densify(sup_rows, sup_cols, sup_vals):
    nnz = sup_rows.shape[0]
    nbuf = (nnz // LANES + 2) * LANES

    mesh = plsc.VectorSubcoreMesh(core_axis_name="c", subcore_axis_name="s")
    fn = functools.partial(
        pl.kernel,
        mesh=mesh,
        out_type=jax.ShapeDtypeStruct((N * N,), jnp.float32),
        scratch_types=[
            pltpu.VMEM((nbuf,), jnp.int32),
            pltpu.VMEM((nbuf,), jnp.int32),
            pltpu.VMEM((nbuf,), jnp.float32),
            pltpu.VMEM((TILE_WORDS,), jnp.float32),
        ],
        compiler_params=pltpu.CompilerParams(needs_layout_passes=False),
    )(functools.partial(_densify_body, nnz))
    return fn(sup_rows.astype(jnp.int32), sup_cols.astype(jnp.int32), sup_vals).reshape(N, N)


def _dotf(a, b):
    return jnp.dot(a, b, preferred_element_type=jnp.float32)


def _dotb(a, b):
    return jnp.dot(a, b, preferred_element_type=jnp.float32).astype(jnp.bfloat16)


def _cell_kernel(s_ref, inp_ref, st_ref, wg_ref, bg_ref, wc_ref, bc_ref, out_ref, out2_ref, sbf_ref):
    @pl.when(pl.program_id(0) == 0)
    def _():
        sbf_ref[...] = s_ref[...].astype(jnp.bfloat16)

    s = sbf_ref[...]
    i0 = inp_ref[0].astype(jnp.bfloat16)
    i1 = inp_ref[1].astype(jnp.bfloat16)
    s0 = st_ref[0]
    s1 = st_ref[1]
    s0b = s0.astype(jnp.bfloat16)
    s1b = s1.astype(jnp.bfloat16)
    ist = jnp.concatenate([i0, i1, s0b, s1b], axis=1)
    d1 = _dotb(s, ist)
    d2 = _dotb(s, d1)
    g_in = jnp.concatenate(
        [
            jnp.concatenate([i0, s0b, d1[:, 0:128], d1[:, 256:384], d2[:, 0:128], d2[:, 256:384]], axis=1),
            jnp.concatenate([i1, s1b, d1[:, 128:256], d1[:, 384:512], d2[:, 128:256], d2[:, 384:512]], axis=1),
        ],
        axis=0,
    )
    g = jax.nn.sigmoid(_dotf(g_in, wg_ref[...]) + bg_ref[0])
    r0 = g[:N, :U]
    u0 = g[:N, U:]
    r1 = g[N:, :U]
    u1 = g[N:, U:]
    rs0 = (r0 * s0).astype(jnp.bfloat16)
    rs1 = (r1 * s1).astype(jnp.bfloat16)
    e1 = _dotb(s, jnp.concatenate([rs0, rs1], axis=1))
    e2 = _dotb(s, e1)
    c_in = jnp.concatenate(
        [
            jnp.concatenate([i0, rs0, d1[:, 0:128], e1[:, 0:128], d2[:, 0:128], e2[:, 0:128]], axis=1),
            jnp.concatenate([i1, rs1, d1[:, 128:256], e1[:, 128:256], d2[:, 128:256], e2[:, 128:256]], axis=1),
        ],
        axis=0,
    )
    c = jnp.tanh(_dotf(c_in, wc_ref[...]) + bc_ref[0])
    o0 = u0 * s0 + (1.0 - u0) * c[:N]
    o1 = u1 * s1 + (1.0 - u1) * c[N:]
    out_ref[0] = o0
    out_ref[1] = o1
    out2_ref[0] = o0
    out2_ref[1] = o1


def _prep_weights(w, out_sz):
    w3 = w.reshape(IN_SZ, NM, out_sz)
    return jnp.concatenate(
        [
            w3[:D_IN, 0] - w3[:D_IN, 2],
            w3[D_IN:, 0] - w3[D_IN:, 2],
            w3[:D_IN, 1],
            w3[D_IN:, 1],
            2.0 * w3[:D_IN, 2],
            2.0 * w3[D_IN:, 2],
        ],
        axis=0,
    )


def kernel(inputs, state, gate_weights, gate_biases, candidate_weights, candidate_biases, sup_rows, sup_cols, sup_vals):
    B = inputs.shape[0]
    BB = 2
    inp = inputs.reshape(B, N, D_IN)
    st = state.reshape(B, N, U)
    wg = _prep_weights(gate_weights, 2 * U)
    wc = _prep_weights(candidate_weights, U)
    bg = gate_biases.reshape(1, 2 * U)
    bc = candidate_biases.reshape(1, U)

    s_dense = _densify(sup_rows, sup_cols, sup_vals)

    out, out2 = pl.pallas_call(
        _cell_kernel,
        grid=(B // BB,),
        in_specs=[
            pl.BlockSpec((N, N), lambda b: (0, 0)),
            pl.BlockSpec((BB, N, D_IN), lambda b: (b, 0, 0)),
            pl.BlockSpec((BB, N, U), lambda b: (b, 0, 0)),
            pl.BlockSpec((NM * IN_SZ, 2 * U), lambda b: (0, 0)),
            pl.BlockSpec((1, 2 * U), lambda b: (0, 0)),
            pl.BlockSpec((NM * IN_SZ, U), lambda b: (0, 0)),
            pl.BlockSpec((1, U), lambda b: (0, 0)),
        ],
        out_specs=[
            pl.BlockSpec((BB, N, U), lambda b: (b, 0, 0)),
            pl.BlockSpec((BB, N, U), lambda b: (b, 0, 0)),
        ],
        out_shape=[
            jax.ShapeDtypeStruct((B, N, U), jnp.float32),
            jax.ShapeDtypeStruct((B, N, U), jnp.float32),
        ],
        scratch_shapes=[pltpu.VMEM((N, N), jnp.bfloat16)],
    )(s_dense, inp, st, wg.astype(jnp.bfloat16), bg, wc.astype(jnp.bfloat16), bc)

    return out.reshape(B, N * U), out2.reshape(B, N * U)


# speedup vs baseline: 589.9329x; 1.0378x over previous
import functools

import jax
import jax.numpy as jnp
from jax import lax
from jax.experimental import pallas as pl
from jax.experimental.pallas import tpu as pltpu
from jax.experimental.pallas import tpu_sc as plsc

N = 512
U = 128
D_IN = 128
IN_SZ = D_IN + U
NM = 3

LANES = 16
NW = 32
ROWS_PER_W = N // NW
TILE_WORDS = ROWS_PER_W * N


def _densify_body(nnz, sr_hbm, sc_hbm, sv_hbm, out_hbm, sr_v, sc_v, sv_v, tile_v):
    wid = lax.axis_index("s") * 2 + lax.axis_index("c")
    lo = wid * ROWS_PER_W
    pltpu.sync_copy(sr_hbm, sr_v)
    pltpu.sync_copy(sc_hbm, sc_v)
    pltpu.sync_copy(sv_hbm, sv_v)

    zeros16 = jnp.zeros((LANES,), jnp.float32)

    def zero_body(i, _):
        tile_v[pl.ds(i * LANES, LANES)] = zeros16
        return 0

    lax.fori_loop(0, TILE_WORDS // LANES, zero_body, 0)

    def lower_bound(target):
        def cond(c):
            return c[0] < c[1]

        def body(c):
            lb, ub = c
            mid = (lb + ub) // 2
            v = sr_v[pl.ds(mid, LANES)][0]
            lt = v < target
            return jnp.where(lt, mid + 1, lb), jnp.where(lt, ub, mid)

        return lax.while_loop(cond, body, (jnp.int32(0), jnp.int32(nnz)))[0]

    e0 = lower_bound(lo)
    e1 = lower_bound(lo + ROWS_PER_W)
    start16 = (e0 // LANES) * LANES
    nvec = (e1 - start16 + LANES - 1) // LANES
    lane = lax.iota(jnp.int32, LANES)

    def edge_body(k, _):
        base = start16 + k * LANES
        r = sr_v[pl.ds(base, LANES)]
        c = sc_v[pl.ds(base, LANES)]
        v = sv_v[pl.ds(base, LANES)]
        m = (r >= lo) & (r < lo + ROWS_PER_W) & (base + lane < nnz)
        lin = (r - lo) * N + c
        plsc.store_scatter(tile_v, [lin], v, mask=m)
        return 0

    lax.fori_loop(0, nvec, edge_body, 0)
    pltpu.sync_copy(tile_v, out_hbm.at[pl.ds(wid * TILE_WORDS, TILE_WORDS)])


def _densify(sup_rows, sup_cols, sup_vals):
    nnz = sup_rows.shape[0]
    nbuf = -(-nnz // 1024) * 1024
    pad = nbuf - nnz
    sup_rows = jnp.concatenate([sup_rows.astype(jnp.int32), jnp.full((pad,), 2 * N, jnp.int32)])
    sup_cols = jnp.concatenate([sup_cols.astype(jnp.int32), jnp.zeros((pad,), jnp.int32)])
    sup_vals = jnp.concatenate([sup_vals, jnp.zeros((pad,), jnp.float32)])

    mesh = plsc.VectorSubcoreMesh(core_axis_name="c", subcore_axis_name="s")
    fn = functools.partial(
        pl.kernel,
        mesh=mesh,
        out_type=jax.ShapeDtypeStruct((N * N,), jnp.float32),
        scratch_types=[
            pltpu.VMEM((nbuf,), jnp.int32),
            pltpu.VMEM((nbuf,), jnp.int32),
            pltpu.VMEM((nbuf,), jnp.float32),
            pltpu.VMEM((TILE_WORDS,), jnp.float32),
        ],
        compiler_params=pltpu.CompilerParams(needs_layout_passes=False),
    )(functools.partial(_densify_body, nnz))
    return fn(sup_rows.astype(jnp.int32), sup_cols.astype(jnp.int32), sup_vals).reshape(N, N)


def _dotf(a, b):
    return jnp.dot(a, b, preferred_element_type=jnp.float32)


def _dotb(a, b):
    return jnp.dot(a, b, preferred_element_type=jnp.float32).astype(jnp.bfloat16)


def _cell_kernel(s_ref, inp_ref, st_ref, wg_ref, bg_ref, wc_ref, bc_ref, out_ref, sbf_ref):
    @pl.when(pl.program_id(0) == 0)
    def _():
        sbf_ref[...] = s_ref[...].astype(jnp.bfloat16)

    s = sbf_ref[...]
    i0 = inp_ref[0].astype(jnp.bfloat16)
    i1 = inp_ref[1].astype(jnp.bfloat16)
    s0 = st_ref[0]
    s1 = st_ref[1]
    s0b = s0.astype(jnp.bfloat16)
    s1b = s1.astype(jnp.bfloat16)
    ist = jnp.concatenate([i0, i1, s0b, s1b], axis=1)
    d1 = _dotb(s, ist)
    d2 = _dotb(s, d1)
    g_in = jnp.concatenate(
        [
            jnp.concatenate([i0, s0b, d1[:, 0:128], d1[:, 256:384], d2[:, 0:128], d2[:, 256:384]], axis=1),
            jnp.concatenate([i1, s1b, d1[:, 128:256], d1[:, 384:512], d2[:, 128:256], d2[:, 384:512]], axis=1),
        ],
        axis=0,
    )
    g = jax.nn.sigmoid(_dotf(g_in, wg_ref[...]) + bg_ref[0])
    r0 = g[:N, :U]
    u0 = g[:N, U:]
    r1 = g[N:, :U]
    u1 = g[N:, U:]
    rs0 = (r0 * s0).astype(jnp.bfloat16)
    rs1 = (r1 * s1).astype(jnp.bfloat16)
    e1 = _dotb(s, jnp.concatenate([rs0, rs1], axis=1))
    e2 = _dotb(s, e1)
    c_in = jnp.concatenate(
        [
            jnp.concatenate([i0, rs0, d1[:, 0:128], e1[:, 0:128], d2[:, 0:128], e2[:, 0:128]], axis=1),
            jnp.concatenate([i1, rs1, d1[:, 128:256], e1[:, 128:256], d2[:, 128:256], e2[:, 128:256]], axis=1),
        ],
        axis=0,
    )
    c = jnp.tanh(_dotf(c_in, wc_ref[...]) + bc_ref[0])
    o0 = u0 * s0 + (1.0 - u0) * c[:N]
    o1 = u1 * s1 + (1.0 - u1) * c[N:]
    out_ref[0] = o0
    out_ref[1] = o1


def _prep_weights(w, out_sz):
    w3 = w.reshape(IN_SZ, NM, out_sz)
    return jnp.concatenate(
        [
            w3[:D_IN, 0] - w3[:D_IN, 2],
            w3[D_IN:, 0] - w3[D_IN:, 2],
            w3[:D_IN, 1],
            w3[D_IN:, 1],
            2.0 * w3[:D_IN, 2],
            2.0 * w3[D_IN:, 2],
        ],
        axis=0,
    )


def kernel(inputs, state, gate_weights, gate_biases, candidate_weights, candidate_biases, sup_rows, sup_cols, sup_vals):
    B = inputs.shape[0]
    BB = 2
    inp = inputs.reshape(B, N, D_IN)
    st = state.reshape(B, N, U)
    wg = _prep_weights(gate_weights, 2 * U)
    wc = _prep_weights(candidate_weights, U)
    bg = gate_biases.reshape(1, 2 * U)
    bc = candidate_biases.reshape(1, U)

    s_dense = _densify(sup_rows, sup_cols, sup_vals)

    out = pl.pallas_call(
        _cell_kernel,
        grid=(B // BB,),
        in_specs=[
            pl.BlockSpec((N, N), lambda b: (0, 0)),
            pl.BlockSpec((BB, N, D_IN), lambda b: (b, 0, 0)),
            pl.BlockSpec((BB, N, U), lambda b: (b, 0, 0)),
            pl.BlockSpec((NM * IN_SZ, 2 * U), lambda b: (0, 0)),
            pl.BlockSpec((1, 2 * U), lambda b: (0, 0)),
            pl.BlockSpec((NM * IN_SZ, U), lambda b: (0, 0)),
            pl.BlockSpec((1, U), lambda b: (0, 0)),
        ],
        out_specs=pl.BlockSpec((BB, N, U), lambda b: (b, 0, 0)),
        out_shape=jax.ShapeDtypeStruct((B, N, U), jnp.float32),
        scratch_shapes=[pltpu.VMEM((N, N), jnp.bfloat16)],
    )(s_dense, inp, st, wg.astype(jnp.bfloat16), bg, wc.astype(jnp.bfloat16), bc)

    y = out.reshape(B, N * U)
    return y, y
